# baseline (speedup 1.0000x reference)
import functools

import jax
import jax.numpy as jnp
from jax import lax
from jax.experimental import pallas as pl
from jax.experimental.pallas import tpu as pltpu
from jax.experimental.pallas import tpu_sc as plsc

_VOCAB = 50257
_N_POS = 2048
_D = 768
_B = 4
_S = 2048
_NW = 32
_SPW = _S // _NW
_LANES = 16
_CHUNK = 16
_NSLOT = 4
_NCHUNK = _B * _SPW // _CHUNK


def _emb_body(ids_hbm, wte_hbm, wpe_hbm, out_hbm, idx_v, wpe_v,
              r0, r1, r2, r3, g0, g1, g2, g3, s0, s1, s2, s3):
    rows = [r0, r1, r2, r3]
    gsem = [g0, g1, g2, g3]
    ssem = [s0, s1, s2, s3]

    cid = lax.axis_index("c")
    sid = lax.axis_index("s")
    wid = sid * 2 + cid
    s_base = wid * _SPW

    idx_copies = [
        pltpu.async_copy(ids_hbm.at[b, pl.ds(s_base, _SPW)], idx_v.at[b],
                         gsem[b])
        for b in range(_B)
    ]
    for cp in idx_copies:
        cp.wait()

    def start_gather(batch, h):
        return pltpu.async_copy(
            wte_hbm.at[idx_v.at[batch, pl.ds(h * _CHUNK, _CHUNK)]],
            rows[h], gsem[h])

    start_gather(0, 0)
    start_gather(0, 1)
    pltpu.sync_copy(wpe_hbm.at[pl.ds(s_base, _SPW)], wpe_v)

    def round_body(r, carry):
        for h in range(_NSLOT):
            c = 4 * r + h
            hp = (h + 2) % _NSLOT
            bp = r + (h + 2) // _NSLOT

            pltpu.make_async_copy(
                wte_hbm.at[idx_v.at[r, pl.ds(h * _CHUNK, _CHUNK)]],
                rows[h], gsem[h]).wait()

            @plsc.parallel_loop(0, _CHUNK, unroll=2)
            def _row_add(rr):
                for col in range(_D // _LANES):
                    sl = pl.ds(col * _LANES, _LANES)
                    plsc.addupdate(rows[h].at[rr, sl],
                                   wpe_v[h * _CHUNK + rr, sl])

            @pl.when(c >= 2)
            def _wait_prev():
                pltpu.make_async_copy(
                    rows[hp], out_hbm.at[0, pl.ds(0, _CHUNK)], ssem[hp]
                ).wait()

            @pl.when(c < _NCHUNK - 2)
            def _prefetch():
                start_gather(bp, hp)

            pltpu.async_copy(
                rows[h], out_hbm.at[r, pl.ds(s_base + h * _CHUNK, _CHUNK)],
                ssem[h])
        return carry

    lax.fori_loop(0, _B, round_body, 0)

    for h in (2, 3):
        pltpu.make_async_copy(
            rows[h], out_hbm.at[0, pl.ds(0, _CHUNK)], ssem[h]).wait()


_emb = functools.partial(
    pl.kernel,
    out_type=jax.ShapeDtypeStruct((_B, _S, _D), jnp.float32),
    mesh=plsc.VectorSubcoreMesh(core_axis_name="c", subcore_axis_name="s"),
    scratch_types=(
        [pltpu.VMEM((_B, _SPW), jnp.int32),
         pltpu.VMEM((_SPW, _D), jnp.float32)]
        + [pltpu.VMEM((_CHUNK, _D), jnp.float32) for _ in range(_NSLOT)]
        + [pltpu.SemaphoreType.DMA for _ in range(2 * _NSLOT)]
    ),
)(_emb_body)


def kernel(input_ids, wte, wpe):
    ids = jnp.asarray(input_ids, jnp.int32)
    return _emb(ids, wte, wpe)


# speedup vs baseline: 1.0429x; 1.0429x over previous
import functools

import jax
import jax.numpy as jnp
from jax import lax
from jax.experimental import pallas as pl
from jax.experimental.pallas import tpu as pltpu
from jax.experimental.pallas import tpu_sc as plsc

_VOCAB = 50257
_N_POS = 2048
_D = 768
_B = 4
_S = 2048
_NW = 32
_SPW = _S // _NW
_LANES = 16
_CHUNK = 16
_NSLOT = 4
_NCHUNK = _B * _SPW // _CHUNK


def _emb_body(ids_hbm, wte_hbm, wpe_hbm, out_hbm, idx_v, wpe_v,
              r0, r1, r2, r3, g0, g1, g2, g3, s0, s1, s2, s3):
    rows = [r0, r1, r2, r3]
    gsem = [g0, g1, g2, g3]
    ssem = [s0, s1, s2, s3]

    cid = lax.axis_index("c")
    sid = lax.axis_index("s")
    wid = sid * 2 + cid
    s_base = wid * _SPW

    idx_copies = [
        pltpu.async_copy(ids_hbm.at[b, pl.ds(s_base, _SPW)], idx_v.at[b],
                         gsem[b])
        for b in range(_B)
    ]
    for cp in idx_copies:
        cp.wait()

    def start_gather(batch, h):
        return pltpu.async_copy(
            wte_hbm.at[idx_v.at[batch, pl.ds(h * _CHUNK, _CHUNK)]],
            rows[h], gsem[h])

    start_gather(0, 0)
    start_gather(0, 1)
    pltpu.sync_copy(wpe_hbm.at[pl.ds(s_base, _SPW)], wpe_v)

    def round_body(r, carry):
        for h in range(_NSLOT):
            c = 4 * r + h
            hp = (h + 2) % _NSLOT
            bp = r + (h + 2) // _NSLOT

            @pl.when(c >= 2)
            def _wait_prev():
                pltpu.make_async_copy(
                    rows[hp], out_hbm.at[0, pl.ds(0, _CHUNK)], ssem[hp]
                ).wait()

            @pl.when(c < _NCHUNK - 2)
            def _prefetch():
                start_gather(bp, hp)

            pltpu.make_async_copy(
                wte_hbm.at[idx_v.at[r, pl.ds(h * _CHUNK, _CHUNK)]],
                rows[h], gsem[h]).wait()

            @plsc.parallel_loop(0, _CHUNK, unroll=2)
            def _row_add(rr):
                for col in range(_D // _LANES):
                    sl = pl.ds(col * _LANES, _LANES)
                    plsc.addupdate(rows[h].at[rr, sl],
                                   wpe_v[h * _CHUNK + rr, sl])

            pltpu.async_copy(
                rows[h], out_hbm.at[r, pl.ds(s_base + h * _CHUNK, _CHUNK)],
                ssem[h])
        return carry

    lax.fori_loop(0, _B, round_body, 0)

    for h in (2, 3):
        pltpu.make_async_copy(
            rows[h], out_hbm.at[0, pl.ds(0, _CHUNK)], ssem[h]).wait()


_emb = functools.partial(
    pl.kernel,
    out_type=jax.ShapeDtypeStruct((_B, _S, _D), jnp.float32),
    mesh=plsc.VectorSubcoreMesh(core_axis_name="c", subcore_axis_name="s"),
    scratch_types=(
        [pltpu.VMEM((_B, _SPW), jnp.int32),
         pltpu.VMEM((_SPW, _D), jnp.float32)]
        + [pltpu.VMEM((_CHUNK, _D), jnp.float32) for _ in range(_NSLOT)]
        + [pltpu.SemaphoreType.DMA for _ in range(2 * _NSLOT)]
    ),
)(_emb_body)


def kernel(input_ids, wte, wpe):
    ids = jnp.asarray(input_ids, jnp.int32)
    return _emb(ids, wte, wpe)


# speedup vs baseline: 1.0576x; 1.0141x over previous
import functools

import jax
import jax.numpy as jnp
from jax import lax
from jax.experimental import pallas as pl
from jax.experimental.pallas import tpu as pltpu
from jax.experimental.pallas import tpu_sc as plsc

_VOCAB = 50257
_N_POS = 2048
_D = 768
_B = 4
_S = 2048
_NW = 32
_SPW = _S // _NW
_LANES = 16
_CHUNK = 16
_NSLOT = 4
_NCHUNK = _B * _SPW // _CHUNK


def _emb_body(ids_hbm, wte_hbm, wpe_hbm, out_hbm, idx_v, wpe_v,
              r0, r1, r2, r3, g0, g1, g2, g3, s0, s1, s2, s3,
              w0, w1, w2, w3):
    rows = [r0, r1, r2, r3]
    gsem = [g0, g1, g2, g3]
    ssem = [s0, s1, s2, s3]
    wsem = [w0, w1, w2, w3]

    cid = lax.axis_index("c")
    sid = lax.axis_index("s")
    wid = sid * 2 + cid
    s_base = wid * _SPW

    idx_copies = [
        pltpu.async_copy(ids_hbm.at[b, pl.ds(s_base, _SPW)], idx_v.at[b],
                         gsem[b])
        for b in range(_B)
    ]
    for cp in idx_copies:
        cp.wait()

    def start_gather(batch, h):
        return pltpu.async_copy(
            wte_hbm.at[idx_v.at[batch, pl.ds(h * _CHUNK, _CHUNK)]],
            rows[h], gsem[h])

    start_gather(0, 0)
    start_gather(0, 1)
    for h in range(_NSLOT):
        pltpu.async_copy(
            wpe_hbm.at[pl.ds(s_base + h * _CHUNK, _CHUNK)],
            wpe_v.at[pl.ds(h * _CHUNK, _CHUNK)], wsem[h])

    def round_body(r, carry):
        for h in range(_NSLOT):
            c = 4 * r + h
            hp = (h + 2) % _NSLOT
            bp = r + (h + 2) // _NSLOT

            @pl.when(c >= 2)
            def _wait_prev():
                pltpu.make_async_copy(
                    rows[hp], out_hbm.at[0, pl.ds(0, _CHUNK)], ssem[hp]
                ).wait()

            @pl.when(c < _NCHUNK - 2)
            def _prefetch():
                start_gather(bp, hp)

            pltpu.make_async_copy(
                wte_hbm.at[idx_v.at[r, pl.ds(h * _CHUNK, _CHUNK)]],
                rows[h], gsem[h]).wait()

            @pl.when(r == 0)
            def _wait_wpe_piece():
                pltpu.make_async_copy(
                    wpe_hbm.at[pl.ds(s_base + h * _CHUNK, _CHUNK)],
                    wpe_v.at[pl.ds(h * _CHUNK, _CHUNK)], wsem[h]).wait()

            @plsc.parallel_loop(0, _CHUNK, unroll=2)
            def _row_add(rr):
                for col in range(_D // _LANES):
                    sl = pl.ds(col * _LANES, _LANES)
                    plsc.addupdate(rows[h].at[rr, sl],
                                   wpe_v[h * _CHUNK + rr, sl])

            pltpu.async_copy(
                rows[h], out_hbm.at[r, pl.ds(s_base + h * _CHUNK, _CHUNK)],
                ssem[h])
        return carry

    lax.fori_loop(0, _B, round_body, 0)

    for h in (2, 3):
        pltpu.make_async_copy(
            rows[h], out_hbm.at[0, pl.ds(0, _CHUNK)], ssem[h]).wait()


_emb = functools.partial(
    pl.kernel,
    out_type=jax.ShapeDtypeStruct((_B, _S, _D), jnp.float32),
    mesh=plsc.VectorSubcoreMesh(core_axis_name="c", subcore_axis_name="s"),
    scratch_types=(
        [pltpu.VMEM((_B, _SPW), jnp.int32),
         pltpu.VMEM((_SPW, _D), jnp.float32)]
        + [pltpu.VMEM((_CHUNK, _D), jnp.float32) for _ in range(_NSLOT)]
        + [pltpu.SemaphoreType.DMA for _ in range(3 * _NSLOT)]
    ),
)(_emb_body)


def kernel(input_ids, wte, wpe):
    ids = jnp.asarray(input_ids, jnp.int32)
    return _emb(ids, wte, wpe)
